# baseline (speedup 1.0000x reference)
import functools

import numpy as np
import jax
import jax.numpy as jnp
from jax import lax
from jax.experimental import pallas as pl
from jax.experimental.pallas import tpu as pltpu
from jax.experimental.pallas import tpu_sc as plsc

_B = 16384
_D = 64
_NBR = 5
_K = _NBR + 1
_NC, _NS, _L = 2, 16, 16
_NW = _NC * _NS
_BPW = _B // _NW
_C = 16
_NCH = _BPW // _C
_DJ = _D // _L
_G = _C * _NBR // _L

def _sc_body(df_t, pos_t, neg_t, it_t,
             g1v_h, g1i_h, g2v_h, g2i_h, w2_h, g3v_h, g3i_h, w3_h, it_h,
             out_h,
             ndf_v, npos_v, nneg_v, tdf_v, tpos_v, tneg_v, tit_v, w2_v, w3_v,
             fdf_v, fpos_v, fneg_v,
             rdf_v, rpos_v, rneg_v, rit_v, obuf_v,
             sem_df, sem_pos, sem_neg, sem_it):
    wid = lax.axis_index("s") * _NC + lax.axis_index("c")
    base0 = pl.multiple_of(wid * _BPW, _BPW)
    sl_w = pl.ds(base0, _BPW)

    for n in range(_NBR):
        sl_src = pl.ds(n * _B + base0, _BPW)
        sl_dst = pl.ds(n * _BPW, _BPW)
        pltpu.sync_copy(g1v_h.at[sl_src], ndf_v.at[sl_dst])
        pltpu.sync_copy(g2v_h.at[sl_src], npos_v.at[sl_dst])
        pltpu.sync_copy(g3v_h.at[sl_src], nneg_v.at[sl_dst])
        pltpu.sync_copy(w2_h.at[sl_src], w2_v.at[sl_dst])
        pltpu.sync_copy(w3_h.at[sl_src], w3_v.at[sl_dst])
    pltpu.sync_copy(g1i_h.at[sl_w], tdf_v)
    pltpu.sync_copy(g2i_h.at[sl_w], tpos_v)
    pltpu.sync_copy(g3i_h.at[sl_w], tneg_v)
    pltpu.sync_copy(it_h.at[sl_w], tit_v)

    def chunk(c, carry):
        base = base0 + c * _C
        sl_c = pl.ds(c * _C, _C)
        for n in range(_NBR):
            src = pl.ds(n * _BPW + c * _C, _L)
            dst = pl.ds(n * _C, _L)
            fdf_v[dst] = ndf_v[src]
            fpos_v[dst] = npos_v[src]
            fneg_v[dst] = nneg_v[src]
        tsl = pl.ds(_NBR * _C, _L)
        fdf_v[tsl] = tdf_v[sl_c]
        fpos_v[tsl] = tpos_v[sl_c]
        fneg_v[tsl] = tneg_v[sl_c]
        cp_df = pltpu.async_copy(df_t.at[fdf_v], rdf_v, sem_df)
        cp_pos = pltpu.async_copy(pos_t.at[fpos_v], rpos_v, sem_pos)
        cp_neg = pltpu.async_copy(neg_t.at[fneg_v], rneg_v, sem_neg)
        cp_it = pltpu.async_copy(it_t.at[tit_v.at[sl_c]], rit_v, sem_it)
        cp_df.wait()
        cp_pos.wait()
        cp_neg.wait()
        cp_it.wait()

        wv2 = [w2_v[pl.ds(n * _BPW + c * _C, _L)] for n in range(_NBR)]
        wv3 = [w3_v[pl.ds(n * _BPW + c * _C, _L)] for n in range(_NBR)]

        for b in range(_C):
            for j in range(_DJ):
                sl = pl.ds(j * _L, _L)
                e = rdf_v[b, sl]
                s = e
                q = e * e
                for n in range(1, _NBR):
                    e = rdf_v[n * _C + b, sl]
                    s = s + e
                    q = q + e * e
                obuf_v[b, sl] = s * s - q + rdf_v[_NBR * _C + b, sl]
                accp = rpos_v[_NBR * _C + b, sl]
                accn = rneg_v[_NBR * _C + b, sl]
                for n in range(_NBR):
                    accp = accp + wv2[n][b] * rpos_v[n * _C + b, sl]
                    accn = accn + wv3[n][b] * rneg_v[n * _C + b, sl]
                obuf_v[b, pl.ds(_D + j * _L, _L)] = accp
                obuf_v[b, pl.ds(2 * _D + j * _L, _L)] = accn
                obuf_v[b, pl.ds(3 * _D + j * _L, _L)] = rit_v[b, sl]

        pltpu.sync_copy(obuf_v, out_h.at[pl.ds(base, _C)])
        return carry

    lax.fori_loop(0, _NCH, chunk, 0)


_sc_gather = functools.partial(
    pl.kernel,
    out_type=jax.ShapeDtypeStruct((_B, 4 * _D), jnp.float32),
    mesh=plsc.VectorSubcoreMesh(core_axis_name="c", subcore_axis_name="s",
                                num_cores=_NC, num_subcores=_NS),
    compiler_params=pltpu.CompilerParams(use_tc_tiling_on_sc=False),
    scratch_types=[
        pltpu.VMEM((_BPW * _NBR,), jnp.int32),
        pltpu.VMEM((_BPW * _NBR,), jnp.int32),
        pltpu.VMEM((_BPW * _NBR,), jnp.int32),
        pltpu.VMEM((_BPW,), jnp.int32),
        pltpu.VMEM((_BPW,), jnp.int32),
        pltpu.VMEM((_BPW,), jnp.int32),
        pltpu.VMEM((_BPW,), jnp.int32),
        pltpu.VMEM((_BPW * _NBR,), jnp.float32),
        pltpu.VMEM((_BPW * _NBR,), jnp.float32),
        pltpu.VMEM((_C * _K,), jnp.int32),
        pltpu.VMEM((_C * _K,), jnp.int32),
        pltpu.VMEM((_C * _K,), jnp.int32),
        pltpu.VMEM((_C * _K, _D), jnp.float32),
        pltpu.VMEM((_C * _K, _D), jnp.float32),
        pltpu.VMEM((_C * _K, _D), jnp.float32),
        pltpu.VMEM((_C, _D), jnp.float32),
        pltpu.VMEM((_C, 4 * _D), jnp.float32),
        pltpu.SemaphoreType.DMA,
        pltpu.SemaphoreType.DMA,
        pltpu.SemaphoreType.DMA,
        pltpu.SemaphoreType.DMA,
    ],
)(_sc_body)


def _mlp_body(x_ref, wq, bq, wk1, bk1, wv1, bv1, wk2, bk2, wv2, bv2,
              wf1, bf1, wf2r, bf2, wf3, bf3, wf4r, bf4,
              wfc1, bfc1, wfc2, bfc2, wfc3r, bfc3, out_ref):
    dot = lambda a, w: lax.dot_general(a.astype(jnp.bfloat16), w,
                                       (((1,), (0,)), ((), ())),
                                       preferred_element_type=jnp.float32)
    x = x_ref[:]
    udf = x[:, 0:_D]
    upos = x[:, _D:2 * _D]
    uneg = x[:, 2 * _D:3 * _D]
    uit = x[:, 3 * _D:4 * _D]
    q = dot(udf, wq[:]) + bq[:]
    k1 = dot(upos, wk1[:]) + bk1[:]
    v1 = dot(upos, wv1[:]) + bv1[:]
    k2 = dot(uneg, wk2[:]) + bk2[:]
    v2 = dot(uneg, wv2[:]) + bv2[:]
    h1 = jnp.maximum(dot(k1 * q, wf1[:]) + bf1[:], 0.0)
    s1 = jax.nn.sigmoid(jnp.sum(h1 * wf2r[:], axis=1, keepdims=True) + bf2[:])
    h2 = jnp.maximum(dot(k2 * q, wf3[:]) + bf3[:], 0.0)
    s2 = jax.nn.sigmoid(jnp.sum(h2 * wf4r[:], axis=1, keepdims=True) + bf4[:])
    users = s1 * v1 + s2 * v2
    a = jnp.maximum(dot(users, wfc1[0:_D, :]) + dot(uit, wfc1[_D:2 * _D, :])
                    + bfc1[:], 0.0)
    a = jnp.maximum(dot(a, wfc2[:]) + bfc2[:], 0.0)
    o = jnp.sum(a * wfc3r[:], axis=1) + bfc3[0, 0]
    out_ref[:] = jax.nn.sigmoid(o)


_BM = 512


def _mlp(x, weights):
    n_blocks = _B // _BM
    full = lambda shp: pl.BlockSpec(shp, lambda i: (0,) * len(shp))
    in_specs = [pl.BlockSpec((_BM, 4 * _D), lambda i: (i, 0))]
    in_specs += [full(w.shape) for w in weights]
    return pl.pallas_call(
        _mlp_body,
        grid=(n_blocks,),
        in_specs=in_specs,
        out_specs=pl.BlockSpec((_BM,), lambda i: (i,)),
        out_shape=jax.ShapeDtypeStruct((_B,), jnp.float32),
    )(x, *weights)


def kernel(u, i, adj_G1_index, adj_G1_values, adj_G2_index, adj_G2_values,
           weights_G2, adj_G3_index, adj_G3_values, weights_G3, params):
    p = params
    i32 = jnp.int32
    f32 = jnp.float32
    bf = jnp.bfloat16

    fl = lambda a, t: a.T.astype(t).reshape(-1)
    one = jnp.float32(1) + u[0].astype(f32) * jnp.float32(1e-38)
    x = _sc_gather(p["users_df"] * one, p["users_pos"] * one,
                   p["users_neg"] * one, p["items"] * one,
                   fl(adj_G1_values, i32), adj_G1_index.astype(i32),
                   fl(adj_G2_values, i32), adj_G2_index.astype(i32),
                   fl(weights_G2, f32),
                   fl(adj_G3_values, i32), adj_G3_index.astype(i32),
                   fl(weights_G3, f32), i.astype(i32))

    r2 = lambda b: b.reshape(1, -1)
    wb = lambda m: m.astype(bf)
    weights = [
        wb(p["query_W"]), r2(p["query_b"]),
        wb(p["key1_W"]), r2(p["key1_b"]), wb(p["value1_W"]), r2(p["value1_b"]),
        wb(p["key2_W"]), r2(p["key2_b"]), wb(p["value2_W"]), r2(p["value2_b"]),
        wb(p["f1_W"]), r2(p["f1_b"]), p["f2_W"].reshape(1, -1), r2(p["f2_b"]),
        wb(p["f3_W"]), r2(p["f3_b"]), p["f4_W"].reshape(1, -1), r2(p["f4_b"]),
        wb(p["fc1_W"]), r2(p["fc1_b"]), wb(p["fc2_W"]), r2(p["fc2_b"]),
        p["fc3_W"].reshape(1, -1), r2(p["fc3_b"]),
    ]
    return _mlp(x, weights)


# speedup vs baseline: 1.3612x; 1.3612x over previous
import functools

import numpy as np
import jax
import jax.numpy as jnp
from jax import lax
from jax.experimental import pallas as pl
from jax.experimental.pallas import tpu as pltpu
from jax.experimental.pallas import tpu_sc as plsc

_B = 16384
_D = 64
_NBR = 5
_K = _NBR + 1
_NC, _NS, _L = 2, 16, 16
_NW = _NC * _NS
_BPW = _B // _NW
_C = 16
_NCH = _BPW // _C
_DJ = _D // _L
_G = _C * _NBR // _L

def _sc_body(df_t, pos_t, neg_t, it_t,
             g1v_h, g1i_h, g2v_h, g2i_h, w2_h, g3v_h, g3i_h, w3_h, it_h,
             out_h,
             ndf_v, npos_v, nneg_v, tdf_v, tpos_v, tneg_v, tit_v, w2_v, w3_v,
             fdf_v, fpos_v, fneg_v,
             rdf_v, rpos_v, rneg_v, rit_v, obuf_v,
             sem_df, sem_pos, sem_neg, sem_it):
    wid = lax.axis_index("s") * _NC + lax.axis_index("c")
    base0 = pl.multiple_of(wid * _BPW, _BPW)
    sl_w = pl.ds(base0, _BPW)

    for n in range(_NBR):
        sl_src = pl.ds(n * _B + base0, _BPW)
        sl_dst = pl.ds(n * _BPW, _BPW)
        pltpu.sync_copy(g1v_h.at[sl_src], ndf_v.at[sl_dst])
        pltpu.sync_copy(g2v_h.at[sl_src], npos_v.at[sl_dst])
        pltpu.sync_copy(g3v_h.at[sl_src], nneg_v.at[sl_dst])
        pltpu.sync_copy(w2_h.at[sl_src], w2_v.at[sl_dst])
        pltpu.sync_copy(w3_h.at[sl_src], w3_v.at[sl_dst])
    pltpu.sync_copy(g1i_h.at[sl_w], tdf_v)
    pltpu.sync_copy(g2i_h.at[sl_w], tpos_v)
    pltpu.sync_copy(g3i_h.at[sl_w], tneg_v)
    pltpu.sync_copy(it_h.at[sl_w], tit_v)

    def chunk(c, carry):
        base = base0 + c * _C
        sl_c = pl.ds(c * _C, _C)
        for n in range(_NBR):
            src = pl.ds(n * _BPW + c * _C, _L)
            dst = pl.ds(n * _C, _L)
            fdf_v[dst] = ndf_v[src]
            fpos_v[dst] = npos_v[src]
            fneg_v[dst] = nneg_v[src]
        tsl = pl.ds(_NBR * _C, _L)
        fdf_v[tsl] = tdf_v[sl_c]
        fpos_v[tsl] = tpos_v[sl_c]
        fneg_v[tsl] = tneg_v[sl_c]
        cp_df = pltpu.async_copy(df_t.at[fdf_v], rdf_v, sem_df)
        cp_pos = pltpu.async_copy(pos_t.at[fpos_v], rpos_v, sem_pos)
        cp_neg = pltpu.async_copy(neg_t.at[fneg_v], rneg_v, sem_neg)
        cp_it = pltpu.async_copy(it_t.at[tit_v.at[sl_c]], rit_v, sem_it)
        cp_df.wait()
        cp_pos.wait()
        cp_neg.wait()
        cp_it.wait()

        wv2 = [w2_v[pl.ds(n * _BPW + c * _C, _L)] for n in range(_NBR)]
        wv3 = [w3_v[pl.ds(n * _BPW + c * _C, _L)] for n in range(_NBR)]

        for b in range(_C):
            for j in range(_DJ):
                sl = pl.ds(j * _L, _L)
                e = rdf_v[b, sl]
                s = e
                q = e * e
                for n in range(1, _NBR):
                    e = rdf_v[n * _C + b, sl]
                    s = s + e
                    q = q + e * e
                obuf_v[b, sl] = s * s - q + rdf_v[_NBR * _C + b, sl]
                accp = rpos_v[_NBR * _C + b, sl]
                accn = rneg_v[_NBR * _C + b, sl]
                for n in range(_NBR):
                    accp = accp + wv2[n][b] * rpos_v[n * _C + b, sl]
                    accn = accn + wv3[n][b] * rneg_v[n * _C + b, sl]
                obuf_v[b, pl.ds(_D + j * _L, _L)] = accp
                obuf_v[b, pl.ds(2 * _D + j * _L, _L)] = accn
                obuf_v[b, pl.ds(3 * _D + j * _L, _L)] = rit_v[b, sl]

        pltpu.sync_copy(obuf_v, out_h.at[pl.ds(base, _C)])
        return carry

    lax.fori_loop(0, _NCH, chunk, 0)


_sc_gather = functools.partial(
    pl.kernel,
    out_type=jax.ShapeDtypeStruct((_B, 4 * _D), jnp.float32),
    mesh=plsc.VectorSubcoreMesh(core_axis_name="c", subcore_axis_name="s",
                                num_cores=_NC, num_subcores=_NS),
    compiler_params=pltpu.CompilerParams(use_tc_tiling_on_sc=False),
    scratch_types=[
        pltpu.VMEM((_BPW * _NBR,), jnp.int32),
        pltpu.VMEM((_BPW * _NBR,), jnp.int32),
        pltpu.VMEM((_BPW * _NBR,), jnp.int32),
        pltpu.VMEM((_BPW,), jnp.int32),
        pltpu.VMEM((_BPW,), jnp.int32),
        pltpu.VMEM((_BPW,), jnp.int32),
        pltpu.VMEM((_BPW,), jnp.int32),
        pltpu.VMEM((_BPW * _NBR,), jnp.float32),
        pltpu.VMEM((_BPW * _NBR,), jnp.float32),
        pltpu.VMEM((_C * _K,), jnp.int32),
        pltpu.VMEM((_C * _K,), jnp.int32),
        pltpu.VMEM((_C * _K,), jnp.int32),
        pltpu.VMEM((_C * _K, _D), jnp.float32),
        pltpu.VMEM((_C * _K, _D), jnp.float32),
        pltpu.VMEM((_C * _K, _D), jnp.float32),
        pltpu.VMEM((_C, _D), jnp.float32),
        pltpu.VMEM((_C, 4 * _D), jnp.float32),
        pltpu.SemaphoreType.DMA,
        pltpu.SemaphoreType.DMA,
        pltpu.SemaphoreType.DMA,
        pltpu.SemaphoreType.DMA,
    ],
)(_sc_body)


def _mlp_body(x_ref, wq, bq, wk1, bk1, wv1, bv1, wk2, bk2, wv2, bv2,
              wf1, bf1, wf2r, bf2, wf3, bf3, wf4r, bf4,
              wfc1, bfc1, wfc2, bfc2, wfc3r, bfc3, out_ref):
    dot = lambda a, w: lax.dot_general(a.astype(jnp.bfloat16), w,
                                       (((1,), (0,)), ((), ())),
                                       preferred_element_type=jnp.float32)
    x = x_ref[:]
    udf = x[:, 0:_D]
    upos = x[:, _D:2 * _D]
    uneg = x[:, 2 * _D:3 * _D]
    uit = x[:, 3 * _D:4 * _D]
    q = dot(udf, wq[:]) + bq[:]
    k1 = dot(upos, wk1[:]) + bk1[:]
    v1 = dot(upos, wv1[:]) + bv1[:]
    k2 = dot(uneg, wk2[:]) + bk2[:]
    v2 = dot(uneg, wv2[:]) + bv2[:]
    h1 = jnp.maximum(dot(k1 * q, wf1[:]) + bf1[:], 0.0)
    s1 = jax.nn.sigmoid(jnp.sum(h1 * wf2r[:], axis=1, keepdims=True) + bf2[:])
    h2 = jnp.maximum(dot(k2 * q, wf3[:]) + bf3[:], 0.0)
    s2 = jax.nn.sigmoid(jnp.sum(h2 * wf4r[:], axis=1, keepdims=True) + bf4[:])
    users = s1 * v1 + s2 * v2
    a = jnp.maximum(dot(users, wfc1[0:_D, :]) + dot(uit, wfc1[_D:2 * _D, :])
                    + bfc1[:], 0.0)
    a = jnp.maximum(dot(a, wfc2[:]) + bfc2[:], 0.0)
    o = jnp.sum(a * wfc3r[:], axis=1) + bfc3[0, 0]
    out_ref[:] = jax.nn.sigmoid(o)


_BM = 512


def _mlp(x, weights):
    n_blocks = _B // _BM
    full = lambda shp: pl.BlockSpec(shp, lambda i: (0,) * len(shp))
    in_specs = [pl.BlockSpec((_BM, 4 * _D), lambda i: (i, 0))]
    in_specs += [full(w.shape) for w in weights]
    return pl.pallas_call(
        _mlp_body,
        grid=(n_blocks,),
        in_specs=in_specs,
        out_specs=pl.BlockSpec((_BM,), lambda i: (i,)),
        out_shape=jax.ShapeDtypeStruct((_B,), jnp.float32),
    )(x, *weights)


def kernel(u, i, adj_G1_index, adj_G1_values, adj_G2_index, adj_G2_values,
           weights_G2, adj_G3_index, adj_G3_values, weights_G3, params):
    p = params
    i32 = jnp.int32
    f32 = jnp.float32
    bf = jnp.bfloat16

    fl = lambda a, t: a.T.astype(t).reshape(-1)
    x = _sc_gather(p["users_df"], p["users_pos"], p["users_neg"], p["items"],
                   fl(adj_G1_values, i32), adj_G1_index.astype(i32),
                   fl(adj_G2_values, i32), adj_G2_index.astype(i32),
                   fl(weights_G2, f32),
                   fl(adj_G3_values, i32), adj_G3_index.astype(i32),
                   fl(weights_G3, f32), i.astype(i32))

    r2 = lambda b: b.reshape(1, -1)
    wb = lambda m: m.astype(bf)
    weights = [
        wb(p["query_W"]), r2(p["query_b"]),
        wb(p["key1_W"]), r2(p["key1_b"]), wb(p["value1_W"]), r2(p["value1_b"]),
        wb(p["key2_W"]), r2(p["key2_b"]), wb(p["value2_W"]), r2(p["value2_b"]),
        wb(p["f1_W"]), r2(p["f1_b"]), p["f2_W"].reshape(1, -1), r2(p["f2_b"]),
        wb(p["f3_W"]), r2(p["f3_b"]), p["f4_W"].reshape(1, -1), r2(p["f4_b"]),
        wb(p["fc1_W"]), r2(p["fc1_b"]), wb(p["fc2_W"]), r2(p["fc2_b"]),
        p["fc3_W"].reshape(1, -1), r2(p["fc3_b"]),
    ]
    return _mlp(x, weights)


# speedup vs baseline: 1.4523x; 1.0669x over previous
import functools

import numpy as np
import jax
import jax.numpy as jnp
from jax import lax
from jax.experimental import pallas as pl
from jax.experimental.pallas import tpu as pltpu
from jax.experimental.pallas import tpu_sc as plsc

_B = 16384
_D = 64
_NBR = 5
_K = _NBR + 1
_NC, _NS, _L = 2, 16, 16
_NW = _NC * _NS
_BPW = _B // _NW
_C = 16
_NCH = _BPW // _C
_DJ = _D // _L
_G = _C * _NBR // _L

def _sc_body(df_t, pos_t, neg_t, it_t,
             g1v_h, g1i_h, g2v_h, g2i_h, w2_h, g3v_h, g3i_h, w3_h, it_h,
             out_h,
             ndf_v, npos_v, nneg_v, tdf_v, tpos_v, tneg_v, tit_v, w2_v, w3_v,
             fdf_v, fpos_v, fneg_v,
             rdf_v, rpos_v, rneg_v, rit_v, obuf_v, wsm_v,
             sem_df, sem_pos, sem_neg, sem_it):
    wid = lax.axis_index("s") * _NC + lax.axis_index("c")
    base0 = pl.multiple_of(wid * _BPW, _BPW)
    sl_w = pl.ds(base0, _BPW)

    for n in range(_NBR):
        sl_src = pl.ds(n * _B + base0, _BPW)
        sl_dst = pl.ds(n * _BPW, _BPW)
        pltpu.sync_copy(g1v_h.at[sl_src], ndf_v.at[sl_dst])
        pltpu.sync_copy(g2v_h.at[sl_src], npos_v.at[sl_dst])
        pltpu.sync_copy(g3v_h.at[sl_src], nneg_v.at[sl_dst])
        pltpu.sync_copy(w2_h.at[sl_src], w2_v.at[sl_dst])
        pltpu.sync_copy(w3_h.at[sl_src], w3_v.at[sl_dst])
    pltpu.sync_copy(g1i_h.at[sl_w], tdf_v)
    pltpu.sync_copy(g2i_h.at[sl_w], tpos_v)
    pltpu.sync_copy(g3i_h.at[sl_w], tneg_v)
    pltpu.sync_copy(it_h.at[sl_w], tit_v)

    def chunk(c, carry):
        base = base0 + c * _C
        sl_c = pl.ds(c * _C, _C)
        for n in range(_NBR):
            src = pl.ds(n * _BPW + c * _C, _L)
            dst = pl.ds(n * _C, _L)
            fdf_v[dst] = ndf_v[src]
            fpos_v[dst] = npos_v[src]
            fneg_v[dst] = nneg_v[src]
        tsl = pl.ds(_NBR * _C, _L)
        fdf_v[tsl] = tdf_v[sl_c]
        fpos_v[tsl] = tpos_v[sl_c]
        fneg_v[tsl] = tneg_v[sl_c]
        cp_df = pltpu.async_copy(df_t.at[fdf_v], rdf_v, sem_df)
        cp_pos = pltpu.async_copy(pos_t.at[fpos_v], rpos_v, sem_pos)
        cp_neg = pltpu.async_copy(neg_t.at[fneg_v], rneg_v, sem_neg)
        cp_it = pltpu.async_copy(it_t.at[tit_v.at[sl_c]], rit_v, sem_it)
        cp_df.wait()
        cp_pos.wait()
        cp_neg.wait()
        cp_it.wait()

        wv2 = [w2_v[pl.ds(n * _BPW + c * _C, _L)] for n in range(_NBR)]
        wv3 = [w3_v[pl.ds(n * _BPW + c * _C, _L)] for n in range(_NBR)]
        for b in range(_C):
            for n in range(_NBR):
                wsm_v[b * 2 * _NBR + n] = wv2[n][b]
                wsm_v[b * 2 * _NBR + _NBR + n] = wv3[n][b]

        def row(b, carry2):
            wbase = b * 2 * _NBR
            for j in range(_DJ):
                sl = pl.ds(j * _L, _L)
                e = rdf_v[b, sl]
                s = e
                q = e * e
                for n in range(1, _NBR):
                    e = rdf_v[n * _C + b, sl]
                    s = s + e
                    q = q + e * e
                obuf_v[b, sl] = s * s - q + rdf_v[_NBR * _C + b, sl]
                accp = rpos_v[_NBR * _C + b, sl]
                accn = rneg_v[_NBR * _C + b, sl]
                for n in range(_NBR):
                    accp = accp + wsm_v[wbase + n] * rpos_v[n * _C + b, sl]
                    accn = accn + wsm_v[wbase + _NBR + n] * rneg_v[n * _C + b, sl]
                obuf_v[b, pl.ds(_D + j * _L, _L)] = accp
                obuf_v[b, pl.ds(2 * _D + j * _L, _L)] = accn
                obuf_v[b, pl.ds(3 * _D + j * _L, _L)] = rit_v[b, sl]
            return carry2

        lax.fori_loop(0, _C, row, 0)
        pltpu.sync_copy(obuf_v, out_h.at[pl.ds(base, _C)])
        return carry

    lax.fori_loop(0, _NCH, chunk, 0)


_sc_gather = functools.partial(
    pl.kernel,
    out_type=jax.ShapeDtypeStruct((_B, 4 * _D), jnp.float32),
    mesh=plsc.VectorSubcoreMesh(core_axis_name="c", subcore_axis_name="s",
                                num_cores=_NC, num_subcores=_NS),
    compiler_params=pltpu.CompilerParams(use_tc_tiling_on_sc=False),
    scratch_types=[
        pltpu.VMEM((_BPW * _NBR,), jnp.int32),
        pltpu.VMEM((_BPW * _NBR,), jnp.int32),
        pltpu.VMEM((_BPW * _NBR,), jnp.int32),
        pltpu.VMEM((_BPW,), jnp.int32),
        pltpu.VMEM((_BPW,), jnp.int32),
        pltpu.VMEM((_BPW,), jnp.int32),
        pltpu.VMEM((_BPW,), jnp.int32),
        pltpu.VMEM((_BPW * _NBR,), jnp.float32),
        pltpu.VMEM((_BPW * _NBR,), jnp.float32),
        pltpu.VMEM((_C * _K,), jnp.int32),
        pltpu.VMEM((_C * _K,), jnp.int32),
        pltpu.VMEM((_C * _K,), jnp.int32),
        pltpu.VMEM((_C * _K, _D), jnp.float32),
        pltpu.VMEM((_C * _K, _D), jnp.float32),
        pltpu.VMEM((_C * _K, _D), jnp.float32),
        pltpu.VMEM((_C, _D), jnp.float32),
        pltpu.VMEM((_C, 4 * _D), jnp.float32),
        pltpu.SMEM((_C * 2 * _NBR,), jnp.float32),
        pltpu.SemaphoreType.DMA,
        pltpu.SemaphoreType.DMA,
        pltpu.SemaphoreType.DMA,
        pltpu.SemaphoreType.DMA,
    ],
)(_sc_body)


def _mlp_body(x_ref, wq, bq, wk1, bk1, wv1, bv1, wk2, bk2, wv2, bv2,
              wf1, bf1, wf2r, bf2, wf3, bf3, wf4r, bf4,
              wfc1, bfc1, wfc2, bfc2, wfc3r, bfc3, out_ref):
    dot = lambda a, w: lax.dot_general(a.astype(jnp.bfloat16), w,
                                       (((1,), (0,)), ((), ())),
                                       preferred_element_type=jnp.float32)
    x = x_ref[:]
    udf = x[:, 0:_D]
    upos = x[:, _D:2 * _D]
    uneg = x[:, 2 * _D:3 * _D]
    uit = x[:, 3 * _D:4 * _D]
    q = dot(udf, wq[:]) + bq[:]
    k1 = dot(upos, wk1[:]) + bk1[:]
    v1 = dot(upos, wv1[:]) + bv1[:]
    k2 = dot(uneg, wk2[:]) + bk2[:]
    v2 = dot(uneg, wv2[:]) + bv2[:]
    h1 = jnp.maximum(dot(k1 * q, wf1[:]) + bf1[:], 0.0)
    s1 = jax.nn.sigmoid(jnp.sum(h1 * wf2r[:], axis=1, keepdims=True) + bf2[:])
    h2 = jnp.maximum(dot(k2 * q, wf3[:]) + bf3[:], 0.0)
    s2 = jax.nn.sigmoid(jnp.sum(h2 * wf4r[:], axis=1, keepdims=True) + bf4[:])
    users = s1 * v1 + s2 * v2
    a = jnp.maximum(dot(users, wfc1[0:_D, :]) + dot(uit, wfc1[_D:2 * _D, :])
                    + bfc1[:], 0.0)
    a = jnp.maximum(dot(a, wfc2[:]) + bfc2[:], 0.0)
    o = jnp.sum(a * wfc3r[:], axis=1) + bfc3[0, 0]
    out_ref[:] = jax.nn.sigmoid(o)


_BM = 512


def _mlp(x, weights):
    n_blocks = _B // _BM
    full = lambda shp: pl.BlockSpec(shp, lambda i: (0,) * len(shp))
    in_specs = [pl.BlockSpec((_BM, 4 * _D), lambda i: (i, 0))]
    in_specs += [full(w.shape) for w in weights]
    return pl.pallas_call(
        _mlp_body,
        grid=(n_blocks,),
        in_specs=in_specs,
        out_specs=pl.BlockSpec((_BM,), lambda i: (i,)),
        out_shape=jax.ShapeDtypeStruct((_B,), jnp.float32),
    )(x, *weights)


def kernel(u, i, adj_G1_index, adj_G1_values, adj_G2_index, adj_G2_values,
           weights_G2, adj_G3_index, adj_G3_values, weights_G3, params):
    p = params
    i32 = jnp.int32
    f32 = jnp.float32
    bf = jnp.bfloat16

    fl = lambda a, t: a.T.astype(t).reshape(-1)
    x = _sc_gather(p["users_df"], p["users_pos"], p["users_neg"], p["items"],
                   fl(adj_G1_values, i32), adj_G1_index.astype(i32),
                   fl(adj_G2_values, i32), adj_G2_index.astype(i32),
                   fl(weights_G2, f32),
                   fl(adj_G3_values, i32), adj_G3_index.astype(i32),
                   fl(weights_G3, f32), i.astype(i32))

    r2 = lambda b: b.reshape(1, -1)
    wb = lambda m: m.astype(bf)
    weights = [
        wb(p["query_W"]), r2(p["query_b"]),
        wb(p["key1_W"]), r2(p["key1_b"]), wb(p["value1_W"]), r2(p["value1_b"]),
        wb(p["key2_W"]), r2(p["key2_b"]), wb(p["value2_W"]), r2(p["value2_b"]),
        wb(p["f1_W"]), r2(p["f1_b"]), p["f2_W"].reshape(1, -1), r2(p["f2_b"]),
        wb(p["f3_W"]), r2(p["f3_b"]), p["f4_W"].reshape(1, -1), r2(p["f4_b"]),
        wb(p["fc1_W"]), r2(p["fc1_b"]), wb(p["fc2_W"]), r2(p["fc2_b"]),
        p["fc3_W"].reshape(1, -1), r2(p["fc3_b"]),
    ]
    return _mlp(x, weights)
